# Initial kernel scaffold; baseline (speedup 1.0000x reference)
import functools

import jax
import jax.numpy as jnp
from jax import lax
from jax.experimental import pallas as pl
from jax.experimental.pallas import tpu as pltpu
from jax.experimental.pallas import tpu_sc as plsc

N = 10000
E = 320000
D = 128

NC = 2
NS = 16
NW = NC * NS

EPW = E // NW
CH = 80
NCHUNK = EPW // CH
RPT = N // NS


def _agg_body(x_hbm, src_hbm, dst_hbm, zeros_hbm, out_hbm,
              acc, src_v, dst_v, rows_v, sem):
    c = lax.axis_index("c")
    s = lax.axis_index("s")
    wid = s * NC + c

    pltpu.sync_copy(zeros_hbm, acc.at[pl.ds(s * RPT, RPT)])
    plsc.subcore_barrier()

    base = wid * EPW

    def body(i, carry):
        off = base + i * CH
        pltpu.sync_copy(src_hbm.at[pl.ds(off, CH)], src_v)
        pltpu.sync_copy(dst_hbm.at[pl.ds(off, CH)], dst_v)
        pltpu.async_copy(x_hbm.at[src_v], rows_v, sem).wait()
        pltpu.sync_copy(rows_v, acc.at[dst_v], add=True)
        return carry

    lax.fori_loop(0, NCHUNK, body, 0)
    plsc.subcore_barrier()

    pltpu.sync_copy(acc.at[pl.ds(s * RPT, RPT)],
                    out_hbm.at[pl.ds(c * N + s * RPT, RPT)])


def _aggregate(x, src, dst, zeros_rows):
    mesh = plsc.VectorSubcoreMesh(core_axis_name="c", subcore_axis_name="s")
    f = pl.kernel(
        _agg_body,
        out_type=jax.ShapeDtypeStruct((2 * N, D), jnp.float32),
        mesh=mesh,
        scratch_types=[
            pltpu.VMEM_SHARED((N, D), jnp.float32),
            pltpu.VMEM((CH,), jnp.int32),
            pltpu.VMEM((CH,), jnp.int32),
            pltpu.VMEM((CH, D), jnp.float32),
            pltpu.SemaphoreType.DMA,
        ],
    )
    return f(x, src, dst, zeros_rows)


def _mlp_body(x_ref, p0_ref, p1_ref, scale_ref, wa_ref, ba_ref, wb_ref,
              bb_ref, o_ref, *, final):
    h = x_ref[...] * scale_ref[...] + p0_ref[...] + p1_ref[...]
    t = jnp.dot(h, wa_ref[...], preferred_element_type=jnp.float32)
    t = jnp.maximum(t + ba_ref[...], 0.0)
    z = jnp.dot(t, wb_ref[...], preferred_element_type=jnp.float32)
    z = z + bb_ref[...]
    if final:
        m = jnp.max(z, axis=1, keepdims=True)
        e = jnp.exp(z - m)
        lse = jnp.log(jnp.sum(e, axis=1, keepdims=True)) + m
        o_ref[...] = z - lse
    else:
        o_ref[...] = jnp.maximum(z, 0.0)


def _mlp(x, p0, p1, scale, waT, ba, wbT, bb, final):
    bn = 1000
    grid = (N // bn,)
    row_spec = pl.BlockSpec((bn, D), lambda i: (i, 0))
    full_spec = pl.BlockSpec((D, D), lambda i: (0, 0))
    vec_spec = pl.BlockSpec((1, D), lambda i: (0, 0))
    return pl.pallas_call(
        functools.partial(_mlp_body, final=final),
        grid=grid,
        in_specs=[row_spec, row_spec, row_spec, vec_spec,
                  full_spec, vec_spec, full_spec, vec_spec],
        out_specs=row_spec,
        out_shape=jax.ShapeDtypeStruct((N, D), jnp.float32),
    )(x, p0, p1, scale, waT, ba, wbT, bb)


def kernel(x, edge_index, eps1, W1a, b1a, W1b, b1b, eps2, W2a, b2a, W2b, b2b):
    src = edge_index[0].astype(jnp.int32)
    dst = edge_index[1].astype(jnp.int32)
    zeros_rows = jnp.zeros((RPT, D), jnp.float32)

    parts1 = _aggregate(x, src, dst, zeros_rows)
    scale1 = jnp.full((1, D), 1.0, jnp.float32) + eps1
    h = _mlp(x, parts1[:N], parts1[N:], scale1,
             W1a.T, b1a.reshape(1, D), W1b.T, b1b.reshape(1, D), final=False)

    parts2 = _aggregate(h, src, dst, zeros_rows)
    scale2 = jnp.full((1, D), 1.0, jnp.float32) + eps2
    out = _mlp(h, parts2[:N], parts2[N:], scale2,
               W2a.T, b2a.reshape(1, D), W2b.T, b2b.reshape(1, D), final=True)
    return out


# speedup vs baseline: 4.5240x; 4.5240x over previous
import functools

import jax
import jax.numpy as jnp
from jax import lax
from jax.experimental import pallas as pl
from jax.experimental.pallas import tpu as pltpu
from jax.experimental.pallas import tpu_sc as plsc

N = 10000
E = 320000
D = 128

NC = 2
NS = 16
NW = NC * NS

EPW = E // NW
CH = 80
NCHUNK = EPW // CH
NPAD = 10240
RPT = NPAD // NS


def _agg_body(x_hbm, src_hbm, dst_hbm, zeros_hbm, out_hbm,
              acc, src_v, dst_v, rows_v, sem):
    c = lax.axis_index("c")
    s = lax.axis_index("s")
    wid = s * NC + c

    pltpu.sync_copy(zeros_hbm, acc.at[pl.ds(s * RPT, RPT)])
    plsc.subcore_barrier()

    base = wid * EPW

    def body(i, carry):
        off = base + i * CH
        pltpu.sync_copy(src_hbm.at[pl.ds(off, CH)], src_v)
        pltpu.sync_copy(dst_hbm.at[pl.ds(off, CH)], dst_v)
        pltpu.async_copy(x_hbm.at[src_v], rows_v, sem).wait()
        pltpu.sync_copy(rows_v, acc.at[dst_v], add=True)
        return carry

    lax.fori_loop(0, NCHUNK, body, 0)
    plsc.subcore_barrier()

    pltpu.sync_copy(acc.at[pl.ds(s * RPT, RPT)],
                    out_hbm.at[pl.ds(c * NPAD + s * RPT, RPT)])


def _aggregate(x, src, dst, zeros_rows):
    mesh = plsc.VectorSubcoreMesh(core_axis_name="c", subcore_axis_name="s")
    f = pl.kernel(
        _agg_body,
        out_type=jax.ShapeDtypeStruct((2 * NPAD, D), jnp.float32),
        mesh=mesh,
        scratch_types=[
            pltpu.VMEM_SHARED((NPAD, D), jnp.float32),
            pltpu.VMEM((CH,), jnp.int32),
            pltpu.VMEM((CH,), jnp.int32),
            pltpu.VMEM((CH, D), jnp.float32),
            pltpu.SemaphoreType.DMA,
        ],
    )
    return f(x, src, dst, zeros_rows)


def _mlp_body(x_ref, p0_ref, p1_ref, scale_ref, wa_ref, ba_ref, wb_ref,
              bb_ref, o_ref, *, final):
    h = x_ref[...] * scale_ref[...] + p0_ref[...] + p1_ref[...]
    t = jnp.dot(h, wa_ref[...], preferred_element_type=jnp.float32)
    t = jnp.maximum(t + ba_ref[...], 0.0)
    z = jnp.dot(t, wb_ref[...], preferred_element_type=jnp.float32)
    z = z + bb_ref[...]
    if final:
        m = jnp.max(z, axis=1, keepdims=True)
        e = jnp.exp(z - m)
        lse = jnp.log(jnp.sum(e, axis=1, keepdims=True)) + m
        o_ref[...] = z - lse
    else:
        o_ref[...] = jnp.maximum(z, 0.0)


def _mlp(x, p0, p1, scale, waT, ba, wbT, bb, final):
    bn = 1000
    grid = (N // bn,)
    row_spec = pl.BlockSpec((bn, D), lambda i: (i, 0))
    full_spec = pl.BlockSpec((D, D), lambda i: (0, 0))
    vec_spec = pl.BlockSpec((1, D), lambda i: (0, 0))
    return pl.pallas_call(
        functools.partial(_mlp_body, final=final),
        grid=grid,
        in_specs=[row_spec, row_spec, row_spec, vec_spec,
                  full_spec, vec_spec, full_spec, vec_spec],
        out_specs=row_spec,
        out_shape=jax.ShapeDtypeStruct((N, D), jnp.float32),
    )(x, p0, p1, scale, waT, ba, wbT, bb)


def kernel(x, edge_index, eps1, W1a, b1a, W1b, b1b, eps2, W2a, b2a, W2b, b2b):
    src = edge_index[0].astype(jnp.int32)
    dst = edge_index[1].astype(jnp.int32)
    zeros_rows = jnp.zeros((RPT, D), jnp.float32)

    parts1 = _aggregate(x, src, dst, zeros_rows)
    scale1 = jnp.full((1, D), 1.0, jnp.float32) + eps1
    h = _mlp(x, parts1[:N], parts1[NPAD:NPAD + N], scale1,
             W1a.T, b1a.reshape(1, D), W1b.T, b1b.reshape(1, D), final=False)

    parts2 = _aggregate(h, src, dst, zeros_rows)
    scale2 = jnp.full((1, D), 1.0, jnp.float32) + eps2
    out = _mlp(h, parts2[:N], parts2[NPAD:NPAD + N], scale2,
               W2a.T, b2a.reshape(1, D), W2b.T, b2b.reshape(1, D), final=True)
    return out


# speedup vs baseline: 8.1353x; 1.7983x over previous
import functools

import jax
import jax.numpy as jnp
from jax import lax
from jax.experimental import pallas as pl
from jax.experimental.pallas import tpu as pltpu
from jax.experimental.pallas import tpu_sc as plsc

N = 10000
E = 320000
D = 128

NC = 2
NS = 16
NW = NC * NS

EPW = E // NW
CH = 80
NCHUNK = EPW // CH
NPAD = 10240
RPT = NPAD // NS


def _agg_body(x_hbm, src_hbm, dst_hbm, zeros_hbm, out_hbm,
              acc, src_v, dst_v, rows0, rows1, g0, g1, isem):
    c = lax.axis_index("c")
    s = lax.axis_index("s")
    wid = s * NC + c

    pltpu.async_copy(src_hbm.at[pl.ds(wid * EPW, EPW)], src_v, isem)
    pltpu.async_copy(dst_hbm.at[wid], dst_v, isem)
    pltpu.sync_copy(zeros_hbm, acc.at[pl.ds(s * RPT, RPT)])
    pltpu.make_async_copy(src_hbm.at[pl.ds(wid * EPW, EPW)], src_v,
                          isem).wait()
    pltpu.make_async_copy(dst_hbm.at[wid], dst_v, isem).wait()
    plsc.subcore_barrier()

    def gather(chunk, rows, sem):
        return pltpu.async_copy(
            x_hbm.at[src_v.at[pl.ds(chunk * CH, CH)]], rows, sem)

    def scat(chunk, rows):
        pltpu.sync_copy(rows, acc.at[dst_v.at[chunk]], add=True)

    gather(0, rows0, g0)

    def body(t, carry):
        c0 = 2 * t
        pltpu.make_async_copy(x_hbm.at[src_v.at[pl.ds(0, CH)]], rows0,
                              g0).wait()
        gather(c0 + 1, rows1, g1)
        scat(c0, rows0)
        pltpu.make_async_copy(x_hbm.at[src_v.at[pl.ds(0, CH)]], rows1,
                              g1).wait()
        gather(c0 + 2, rows0, g0)
        scat(c0 + 1, rows1)
        return carry

    lax.fori_loop(0, (NCHUNK - 1) // 2, body, 0)
    pltpu.make_async_copy(x_hbm.at[src_v.at[pl.ds(0, CH)]], rows0, g0).wait()
    scat(NCHUNK - 1, rows0)

    plsc.subcore_barrier()
    pltpu.sync_copy(acc.at[pl.ds(s * RPT, RPT)],
                    out_hbm.at[pl.ds(c * NPAD + s * RPT, RPT)])


def _aggregate(x, src, dst3d, zeros_rows):
    mesh = plsc.VectorSubcoreMesh(core_axis_name="c", subcore_axis_name="s")
    f = pl.kernel(
        _agg_body,
        out_type=jax.ShapeDtypeStruct((2 * NPAD, D), jnp.float32),
        mesh=mesh,
        scratch_types=[
            pltpu.VMEM_SHARED((NPAD, D), jnp.float32),
            pltpu.VMEM((EPW,), jnp.int32),
            pltpu.VMEM((NCHUNK, CH), jnp.int32),
            pltpu.VMEM((CH, D), jnp.float32),
            pltpu.VMEM((CH, D), jnp.float32),
            pltpu.SemaphoreType.DMA,
            pltpu.SemaphoreType.DMA,
            pltpu.SemaphoreType.DMA,
        ],
    )
    return f(x, src, dst3d, zeros_rows)


def _mlp_body(x_ref, p0_ref, p1_ref, scale_ref, wa_ref, ba_ref, wb_ref,
              bb_ref, o_ref, *, final):
    h = x_ref[...] * scale_ref[...] + p0_ref[...] + p1_ref[...]
    t = jnp.dot(h, wa_ref[...], preferred_element_type=jnp.float32)
    t = jnp.maximum(t + ba_ref[...], 0.0)
    z = jnp.dot(t, wb_ref[...], preferred_element_type=jnp.float32)
    z = z + bb_ref[...]
    if final:
        m = jnp.max(z, axis=1, keepdims=True)
        e = jnp.exp(z - m)
        lse = jnp.log(jnp.sum(e, axis=1, keepdims=True)) + m
        o_ref[...] = z - lse
    else:
        o_ref[...] = jnp.maximum(z, 0.0)


def _mlp(x, p0, p1, scale, waT, ba, wbT, bb, final):
    bn = 1000
    grid = (N // bn,)
    row_spec = pl.BlockSpec((bn, D), lambda i: (i, 0))
    full_spec = pl.BlockSpec((D, D), lambda i: (0, 0))
    vec_spec = pl.BlockSpec((1, D), lambda i: (0, 0))
    return pl.pallas_call(
        functools.partial(_mlp_body, final=final),
        grid=grid,
        in_specs=[row_spec, row_spec, row_spec, vec_spec,
                  full_spec, vec_spec, full_spec, vec_spec],
        out_specs=row_spec,
        out_shape=jax.ShapeDtypeStruct((N, D), jnp.float32),
    )(x, p0, p1, scale, waT, ba, wbT, bb)


def kernel(x, edge_index, eps1, W1a, b1a, W1b, b1b, eps2, W2a, b2a, W2b, b2b):
    src = edge_index[0].astype(jnp.int32)
    dst = edge_index[1].astype(jnp.int32)
    zeros_rows = jnp.zeros((RPT, D), jnp.float32)

    dst3d = dst.reshape(NW, NCHUNK, CH)
    parts1 = _aggregate(x, src, dst3d, zeros_rows)
    scale1 = jnp.full((1, D), 1.0, jnp.float32) + eps1
    h = _mlp(x, parts1[:N], parts1[NPAD:NPAD + N], scale1,
             W1a.T, b1a.reshape(1, D), W1b.T, b1b.reshape(1, D), final=False)

    parts2 = _aggregate(h, src, dst3d, zeros_rows)
    scale2 = jnp.full((1, D), 1.0, jnp.float32) + eps2
    out = _mlp(h, parts2[:N], parts2[NPAD:NPAD + N], scale2,
               W2a.T, b2a.reshape(1, D), W2b.T, b2b.reshape(1, D), final=True)
    return out
